# baseline (speedup 1.0000x reference)
import jax
import jax.numpy as jnp
from jax import lax
from jax.experimental import pallas as pl
from jax.experimental.pallas import tpu as pltpu
from jax.experimental.pallas import tpu_sc as plsc

D = 768
B = 1024
SEQ = 77
CA = 16
REST = SEQ - CA - 1
NTOK = 1 + REST
L = 16
TKR = 64

_info = plsc.get_sparse_core_info()
_NC, _NS = _info.num_cores, _info.num_subcores
NW = _NC * _NS
NB = B // NW
REP = 16
NSLOT = 4


def _body(tokt_hbm, emb_hbm, ca_hbm, out_hbm,
          tki, rep, g0, g1, g2, g3,
          sg0, sg1, sg2, sg3, sw0, sw1, sw2, sw3, sr, sc):
    wid = lax.axis_index("s") * _NC + lax.axis_index("c")
    base = pl.multiple_of(wid * NB, NB)
    gcol = pl.multiple_of((wid // 4) * 128, 128)
    col0 = pl.multiple_of((wid % 4) * NB, NB)
    pltpu.sync_copy(tokt_hbm.at[pl.ds(0, TKR), pl.ds(gcol, 128)], tki)

    bufs = ((g0, sg0, sw0), (g1, sg1, sw1), (g2, sg2, sw2), (g3, sg3, sw3))

    def start(k, gbuf, sg):
        iv0 = tki[k, pl.ds(col0, L)]
        iv1 = tki[k, pl.ds(col0 + L, L)]
        pltpu.async_copy(emb_hbm.at[iv0], gbuf.at[pl.ds(0, L)], sg)
        pltpu.async_copy(emb_hbm.at[iv1], gbuf.at[pl.ds(L, L)], sg)

    for s in range(NSLOT - 1):
        start(s, bufs[s][0], bufs[s][1])

    i = wid % CA
    half = wid // CA
    civ = jnp.full((L,), i, dtype=jnp.int32)
    pltpu.async_copy(ca_hbm.at[civ], rep, sr)
    pltpu.make_async_copy(ca_hbm.at[pl.ds(0, REP)], rep, sr).wait()
    cbase = pl.multiple_of(half * (B // 2), B // 2)
    NCW = B // 2 // REP
    for j in range(NCW):
        pltpu.async_copy(rep, out_hbm.at[i + 1, pl.ds(cbase + j * REP, REP)], sc)

    def dst(kk):
        s_out = jnp.where(kk == 0, 0, kk + CA)
        return out_hbm.at[s_out, pl.ds(base, NB)]

    def ring(p, carry):
        k = NSLOT * p
        for slot in range(NSLOT):
            kk = k + slot
            gbuf, sg, sw = bufs[slot]

            @pl.when(kk >= 1)
            def _drain_prev():
                pb, _, psw = bufs[(slot - 1) % NSLOT]
                pltpu.make_async_copy(pb, dst(kk - 1), psw).wait()

            pltpu.make_async_copy(emb_hbm.at[pl.ds(0, NB)], gbuf, sg).wait()
            pltpu.async_copy(gbuf, dst(kk), sw)

            @pl.when(kk + NSLOT - 1 < NTOK)
            def _start_next():
                nb_, nsg, _ = bufs[(slot - 1) % NSLOT]
                start(kk + NSLOT - 1, nb_, nsg)

        return carry

    lax.fori_loop(0, NTOK // NSLOT, ring, 0)

    slot = (NTOK - 1) % NSLOT
    gbuf, sg, sw = bufs[slot]
    pb, _, psw = bufs[(slot - 1) % NSLOT]
    pltpu.make_async_copy(pb, dst(NTOK - 2), psw).wait()
    pltpu.make_async_copy(emb_hbm.at[pl.ds(0, NB)], gbuf, sg).wait()
    pltpu.sync_copy(gbuf, dst(NTOK - 1))
    for j in range(NCW):
        pltpu.make_async_copy(rep, out_hbm.at[i + 1, pl.ds(cbase, REP)], sc).wait()


def kernel(tokenized_text, dynamic_bools, token_embedding, ca_vectors, da_vectors):
    tokt = tokenized_text.astype(jnp.int32).T
    mesh = plsc.VectorSubcoreMesh(core_axis_name="c", subcore_axis_name="s")
    f = pl.kernel(
        _body,
        mesh=mesh,
        compiler_params=pltpu.CompilerParams(use_tc_tiling_on_sc=True),
        out_type=jax.ShapeDtypeStruct((SEQ, B, D), jnp.float32),
        scratch_types=(
            [pltpu.VMEM((TKR, 128), jnp.int32),
             pltpu.VMEM((REP, D), jnp.float32)]
            + [pltpu.VMEM((NB, D), jnp.float32)] * NSLOT
            + [pltpu.SemaphoreType.DMA] * (2 * NSLOT + 2)
        ),
    )
    out_t = f(tokt, token_embedding, ca_vectors)
    return jnp.transpose(out_t, (1, 0, 2))


# speedup vs baseline: 1.0069x; 1.0069x over previous
import jax
import jax.numpy as jnp
from jax import lax
from jax.experimental import pallas as pl
from jax.experimental.pallas import tpu as pltpu
from jax.experimental.pallas import tpu_sc as plsc

D = 768
B = 1024
SEQ = 77
CA = 16
REST = SEQ - CA - 1
NTOK = 1 + REST
L = 16
TKR = 64

_info = plsc.get_sparse_core_info()
_NC, _NS = _info.num_cores, _info.num_subcores
NW = _NC * _NS
NB = B // NW
REP = 32
NSLOT = 3


def _body(tokt_hbm, emb_hbm, ca_hbm, out_hbm,
          tki, rep, g0, g1, g2, sg0, sg1, sg2, sr):
    wid = lax.axis_index("s") * _NC + lax.axis_index("c")
    base = pl.multiple_of(wid * NB, NB)
    gcol = pl.multiple_of((wid // 4) * 128, 128)
    col0 = pl.multiple_of((wid % 4) * NB, NB)
    pltpu.sync_copy(tokt_hbm.at[pl.ds(0, TKR), pl.ds(gcol, 128)], tki)

    bufs = ((g0, sg0), (g1, sg1), (g2, sg2))

    def start(k, gbuf, sg):
        iv0 = tki[k, pl.ds(col0, L)]
        iv1 = tki[k, pl.ds(col0 + L, L)]
        pltpu.async_copy(emb_hbm.at[iv0], gbuf.at[pl.ds(0, L)], sg)
        pltpu.async_copy(emb_hbm.at[iv1], gbuf.at[pl.ds(L, L)], sg)

    for s in range(NSLOT):
        start(s, *bufs[s])

    i = wid % CA
    half = wid // CA
    civ = jnp.full((L,), i, dtype=jnp.int32)
    for j in range(REP // L):
        pltpu.async_copy(ca_hbm.at[civ], rep.at[pl.ds(j * L, L)], sr)
    pltpu.make_async_copy(ca_hbm.at[pl.ds(0, REP)], rep, sr).wait()
    cbase = pl.multiple_of(half * (B // 2), B // 2)
    for j in range(B // 2 // REP):
        pltpu.sync_copy(rep, out_hbm.at[i + 1, pl.ds(cbase + j * REP, REP)])

    def triple(p, carry):
        k = NSLOT * p
        for slot in range(NSLOT):
            kk = k + slot
            gbuf, sg = bufs[slot]
            pltpu.make_async_copy(emb_hbm.at[pl.ds(0, NB)], gbuf, sg).wait()
            s_out = jnp.where(kk == 0, 0, kk + CA)
            pltpu.sync_copy(gbuf, out_hbm.at[s_out, pl.ds(base, NB)])

            @pl.when(kk + NSLOT < NTOK)
            def _start_next():
                start(kk + NSLOT, gbuf, sg)

        return carry

    lax.fori_loop(0, REST // NSLOT, triple, 0)

    gbuf, sg = bufs[0]
    pltpu.make_async_copy(emb_hbm.at[pl.ds(0, NB)], gbuf, sg).wait()
    pltpu.sync_copy(gbuf, out_hbm.at[NTOK - 1 + CA, pl.ds(base, NB)])


def kernel(tokenized_text, dynamic_bools, token_embedding, ca_vectors, da_vectors):
    tokt = tokenized_text.astype(jnp.int32).T
    mesh = plsc.VectorSubcoreMesh(core_axis_name="c", subcore_axis_name="s")
    f = pl.kernel(
        _body,
        mesh=mesh,
        compiler_params=pltpu.CompilerParams(use_tc_tiling_on_sc=True),
        out_type=jax.ShapeDtypeStruct((SEQ, B, D), jnp.float32),
        scratch_types=[
            pltpu.VMEM((TKR, 128), jnp.int32),
            pltpu.VMEM((REP, D), jnp.float32),
            pltpu.VMEM((NB, D), jnp.float32),
            pltpu.VMEM((NB, D), jnp.float32),
            pltpu.VMEM((NB, D), jnp.float32),
            pltpu.SemaphoreType.DMA,
            pltpu.SemaphoreType.DMA,
            pltpu.SemaphoreType.DMA,
            pltpu.SemaphoreType.DMA,
        ],
    )
    out_t = f(tokt, token_embedding, ca_vectors)
    return jnp.transpose(out_t, (1, 0, 2))
